# baseline (speedup 1.0000x reference)
import jax
import jax.numpy as jnp
from jax import lax
from jax.experimental import pallas as pl
from jax.experimental.pallas import tpu as pltpu
from jax.experimental.pallas import tpu_sc as plsc

NC = 2
NS = 16
NW = NC * NS
L = 16

D = 64
V = 1000000
NVT = V // 128
VT_MAIN = NVT - (NVT % NW)
B = 4096
H = 200
HB = 8
NBT = B // 128
N_BLK = (H // HB) * NBT


def _detile_kernel():
    mesh = plsc.VectorSubcoreMesh(core_axis_name="c", subcore_axis_name="s")
    per_w = VT_MAIN // NW
    UNROLL = 4
    T = per_w // UNROLL

    def body(tt_hbm, tail_hbm, out_hbm, strip_v, rows_v, tail_v, ssem, osem):
        wid = lax.axis_index("s") * NC + lax.axis_index("c")
        base = wid * per_w

        def start_strip(vt, s):
            for dt in range(8):
                pltpu.make_async_copy(
                    tt_hbm.at[pl.ds(dt * 8, 8), pl.ds(vt * 128, 128)],
                    strip_v.at[s, pl.ds(dt * 8, 8)], ssem).start()

        def wait_strip():
            pltpu.make_async_copy(
                tt_hbm.at[pl.ds(0, 64), pl.ds(0, 128)],
                strip_v.at[0], ssem).wait()

        def o_copy(vt, s):
            return pltpu.make_async_copy(
                rows_v.at[s], out_hbm.at[pl.ds(vt * 64, 64)], osem)

        def o_drain():
            pltpu.make_async_copy(
                rows_v.at[0], out_hbm.at[pl.ds(0, 64)], osem).wait()

        def transpose(s):
            def prow(p, _):
                for half in range(2):
                    rcol = jnp.full((L,), p * 2 + half, jnp.int32)
                    for q in range(D // L):
                        drows = jnp.arange(q * L, q * L + L, dtype=jnp.int32)
                        rows_v[s, p, pl.ds(half * D + q * L, L)] = (
                            plsc.load_gather(strip_v.at[s], [drows, rcol]))
                return 0

            lax.fori_loop(0, 64, prow, 0)

        start_strip(base, 0)

        def step(t, _):
            for b in range(UNROLL):
                i = base + t * UNROLL + b
                sn = (b + 1) % UNROLL
                if b < UNROLL - 1:
                    @pl.when(t > 0)
                    def _w():
                        o_drain()

                    start_strip(i + 1, sn)
                else:
                    @pl.when(t + 1 < T)
                    def _wr():
                        o_drain()
                        start_strip(i + 1, sn)

                wait_strip()
                transpose(b)
                o_copy(i, b).start()
            return 0

        lax.fori_loop(0, T, step, 0)
        for _ in range(UNROLL):
            o_drain()

        @pl.when(wid < NVT - VT_MAIN)
        def _leftover():
            vt = VT_MAIN + wid
            start_strip(vt, 0)
            wait_strip()
            transpose(0)
            c = o_copy(vt, 0)
            c.start()
            c.wait()

        @pl.when(wid == NW - 1)
        def _tail():
            pltpu.sync_copy(tail_hbm, tail_v)
            pltpu.sync_copy(tail_v, out_hbm.at[pl.ds(NVT * 64, 32)])

    return pl.kernel(
        body,
        out_type=jax.ShapeDtypeStruct((V // 2, 128), jnp.float32),
        mesh=mesh,
        compiler_params=pltpu.CompilerParams(
            needs_layout_passes=False, use_tc_tiling_on_sc=True),
        scratch_types=[
            pltpu.VMEM((4, 64, 128), jnp.float32),
            pltpu.VMEM((4, 64, 128), jnp.float32),
            pltpu.VMEM((32, 128), jnp.float32),
            pltpu.SemaphoreType.DMA,
            pltpu.SemaphoreType.DMA,
        ],
    )


def _gather_kernel():
    mesh = plsc.VectorSubcoreMesh(core_axis_name="c", subcore_axis_name="s")
    n_per_w = N_BLK // NW

    def body(tab_hbm, idxt_hbm, out_hbm, idx_v, krows_v, g_v, o_v,
             isem, gsem, osem):
        wid = lax.axis_index("s") * NC + lax.axis_index("c")

        def item(n):
            j = wid * n_per_w + n
            hb = j // NBT
            bt = j - hb * NBT
            return hb * HB, bt * 128

        def load_idx(n, bp):
            h0, b0 = item(n)
            c = pltpu.make_async_copy(
                idxt_hbm.at[pl.ds(h0, HB), pl.ds(b0, 128)], idx_v.at[bp],
                isem)
            c.start()
            c.wait()

        def start_gather(bp, hh, s):
            for g in range(8):
                i16 = idx_v[bp, hh, pl.ds(g * L, L)]
                krows_v[s, pl.ds(g * L, L)] = jnp.right_shift(i16, 1)
            pltpu.make_async_copy(
                tab_hbm.at[krows_v.at[s]], g_v.at[s], gsem).start()

        def wait_gather():
            pltpu.make_async_copy(
                tab_hbm.at[krows_v.at[0]], g_v.at[0], gsem).wait()

        def o_copy(n, hh, s):
            h0, b0 = item(n)
            return pltpu.make_async_copy(
                o_v.at[s], out_hbm.at[h0 + hh, pl.ds(0, D), pl.ds(b0, 128)],
                osem)

        def o_drain():
            pltpu.make_async_copy(
                o_v.at[0], out_hbm.at[0, pl.ds(0, D), pl.ds(0, 128)],
                osem).wait()

        def transpose(bp, hh, s):
            for g in range(8):
                i16 = idx_v[bp, hh, pl.ds(g * L, L)]
                rows16 = jnp.arange(g * L, g * L + L, dtype=jnp.int32)
                cbase = (i16 & 1) * D
                m = (i16 != 0).astype(jnp.float32)

                def drow(dd, _):
                    for u in range(4):
                        d = dd * 4 + u
                        col = plsc.load_gather(
                            g_v.at[s], [rows16, cbase + d]) * m
                        o_v[s, d, pl.ds(g * L, L)] = col
                    return 0

                lax.fori_loop(0, D // 4, drow, 0)

        load_idx(0, 0)
        start_gather(0, 0, 0)

        def blk(n, _):
            bp = n & 1
            bpn = 1 - bp
            for hh in range(HB):
                s = hh % 4
                sn = (hh + 1) % 4
                if hh < HB - 1:
                    if hh >= 3:
                        o_drain()
                    else:
                        @pl.when(n > 0)
                        def _w():
                            o_drain()
                    start_gather(bp, hh + 1, sn)
                else:
                    @pl.when(n + 1 < n_per_w)
                    def _adv():
                        o_drain()
                        load_idx(n + 1, bpn)
                        start_gather(bpn, 0, sn)

                wait_gather()
                transpose(bp, hh, s)
                o_copy(n, hh, s).start()
            return 0

        lax.fori_loop(0, n_per_w, blk, 0)
        for _ in range(4):
            o_drain()

    return pl.kernel(
        body,
        out_type=jax.ShapeDtypeStruct((H, D, B), jnp.float32),
        mesh=mesh,
        compiler_params=pltpu.CompilerParams(
            needs_layout_passes=False, use_tc_tiling_on_sc=True),
        scratch_types=[
            pltpu.VMEM((2, HB, 128), jnp.int32),
            pltpu.VMEM((4, 128), jnp.int32),
            pltpu.VMEM((4, 128, 128), jnp.float32),
            pltpu.VMEM((4, D, 128), jnp.float32),
            pltpu.SemaphoreType.DMA,
            pltpu.SemaphoreType.DMA,
            pltpu.SemaphoreType.DMA,
        ],
    )


def kernel(price_hist, price_lens, tweet_hist, tweet_lens, embedding_matrix):
    tt = embedding_matrix.T
    tail = lax.slice(embedding_matrix, (NVT * 128, 0), (V, D)).reshape(32, 128)
    table2 = _detile_kernel()(tt, tail)
    idxt = tweet_hist.astype(jnp.int32).T
    out_t = _gather_kernel()(table2, idxt)
    return out_t.transpose(2, 0, 1)


# speedup vs baseline: 1.9190x; 1.9190x over previous
import jax
import jax.numpy as jnp
from jax import lax
from jax.experimental import pallas as pl
from jax.experimental.pallas import tpu as pltpu
from jax.experimental.pallas import tpu_sc as plsc

NC = 2
NS = 16
NW = NC * NS
L = 16

D = 64
V = 1000000
NVT = V // 128
VT_MAIN = NVT - (NVT % NW)
B = 4096
H = 200
HB = 8
NBT = B // 128
N_BLK = (H // HB) * NBT


def _detile_kernel():
    mesh = plsc.VectorSubcoreMesh(core_axis_name="c", subcore_axis_name="s")
    per_w = VT_MAIN // NW
    UNROLL = 4
    T = per_w // UNROLL

    def body(tt_hbm, tail_hbm, out_hbm, strip_v, rows_v, tail_v, ssem, osem):
        wid = lax.axis_index("s") * NC + lax.axis_index("c")
        base = wid * per_w

        def start_strip(vt, s):
            pltpu.make_async_copy(
                tt_hbm.at[pl.ds(0, 64), pl.ds(vt * 128, 128)],
                strip_v.at[s], ssem).start()

        def wait_strip():
            pltpu.make_async_copy(
                tt_hbm.at[pl.ds(0, 64), pl.ds(0, 128)],
                strip_v.at[0], ssem).wait()

        def o_copy(vt, s):
            return pltpu.make_async_copy(
                rows_v.at[s], out_hbm.at[pl.ds(vt * 64, 64)], osem)

        def o_drain():
            pltpu.make_async_copy(
                rows_v.at[0], out_hbm.at[pl.ds(0, 64)], osem).wait()

        def transpose(s):
            @plsc.parallel_loop(0, 64, unroll=4)
            def _prow(p):
                for half in range(2):
                    rcol = jnp.full((L,), p * 2 + half, jnp.int32)
                    for q in range(D // L):
                        drows = jnp.arange(q * L, q * L + L, dtype=jnp.int32)
                        rows_v[s, p, pl.ds(half * D + q * L, L)] = (
                            plsc.load_gather(strip_v.at[s], [drows, rcol]))

        start_strip(base, 0)

        def step(t, _):
            for b in range(UNROLL):
                i = base + t * UNROLL + b
                sn = (b + 1) % UNROLL
                if b < UNROLL - 1:
                    @pl.when(t > 0)
                    def _w():
                        o_drain()

                    start_strip(i + 1, sn)
                else:
                    @pl.when(t + 1 < T)
                    def _wr():
                        o_drain()
                        start_strip(i + 1, sn)

                wait_strip()
                transpose(b)
                o_copy(i, b).start()
            return 0

        lax.fori_loop(0, T, step, 0)
        for _ in range(UNROLL):
            o_drain()

        @pl.when(wid < NVT - VT_MAIN)
        def _leftover():
            vt = VT_MAIN + wid
            start_strip(vt, 0)
            wait_strip()
            transpose(0)
            c = o_copy(vt, 0)
            c.start()
            c.wait()

        @pl.when(wid == NW - 1)
        def _tail():
            pltpu.sync_copy(tail_hbm, tail_v)
            pltpu.sync_copy(tail_v, out_hbm.at[pl.ds(NVT * 64, 32)])

    return pl.kernel(
        body,
        out_type=jax.ShapeDtypeStruct((V // 2, 128), jnp.float32),
        mesh=mesh,
        compiler_params=pltpu.CompilerParams(
            needs_layout_passes=False, use_tc_tiling_on_sc=True),
        scratch_types=[
            pltpu.VMEM((4, 64, 128), jnp.float32),
            pltpu.VMEM((4, 64, 128), jnp.float32),
            pltpu.VMEM((32, 128), jnp.float32),
            pltpu.SemaphoreType.DMA,
            pltpu.SemaphoreType.DMA,
        ],
    )


def _gather_kernel():
    mesh = plsc.VectorSubcoreMesh(core_axis_name="c", subcore_axis_name="s")
    n_per_w = N_BLK // NW

    def body(tab_hbm, idxt_hbm, out_hbm, idx_v, krows_v, g_v, o_v,
             isem, gsem, osem):
        wid = lax.axis_index("s") * NC + lax.axis_index("c")

        def item(n):
            j = wid * n_per_w + n
            hb = j // NBT
            bt = j - hb * NBT
            return hb * HB, bt * 128

        def load_idx(n, bp):
            h0, b0 = item(n)
            c = pltpu.make_async_copy(
                idxt_hbm.at[pl.ds(h0, HB), pl.ds(b0, 128)], idx_v.at[bp],
                isem)
            c.start()
            c.wait()

        def start_gather(bp, hh, s):
            for g in range(8):
                i16 = idx_v[bp, hh, pl.ds(g * L, L)]
                krows_v[s, pl.ds(g * L, L)] = jnp.right_shift(i16, 1)
            pltpu.make_async_copy(
                tab_hbm.at[krows_v.at[s]], g_v.at[s], gsem).start()

        def wait_gather():
            pltpu.make_async_copy(
                tab_hbm.at[krows_v.at[0]], g_v.at[0], gsem).wait()

        def o_copy(n, hh, s):
            h0, b0 = item(n)
            return pltpu.make_async_copy(
                o_v.at[s], out_hbm.at[h0 + hh, pl.ds(0, D), pl.ds(b0, 128)],
                osem)

        def o_drain():
            pltpu.make_async_copy(
                o_v.at[0], out_hbm.at[0, pl.ds(0, D), pl.ds(0, 128)],
                osem).wait()

        def transpose(bp, hh, s):
            for g in range(8):
                i16 = idx_v[bp, hh, pl.ds(g * L, L)]
                rows16 = jnp.arange(g * L, g * L + L, dtype=jnp.int32)
                cbase = (i16 & 1) * D
                m = (i16 != 0).astype(jnp.float32)

                @plsc.parallel_loop(0, D, unroll=8)
                def _drow(d):
                    o_v[s, d, pl.ds(g * L, L)] = plsc.load_gather(
                        g_v.at[s], [rows16, cbase + d]) * m

        load_idx(0, 0)
        start_gather(0, 0, 0)

        def blk(n, _):
            bp = n & 1
            bpn = 1 - bp
            for hh in range(HB):
                s = hh % 4
                sn = (hh + 1) % 4
                if hh < HB - 1:
                    if hh >= 3:
                        o_drain()
                    else:
                        @pl.when(n > 0)
                        def _w():
                            o_drain()
                    start_gather(bp, hh + 1, sn)
                else:
                    @pl.when(n + 1 < n_per_w)
                    def _adv():
                        o_drain()
                        load_idx(n + 1, bpn)
                        start_gather(bpn, 0, sn)

                wait_gather()
                transpose(bp, hh, s)
                o_copy(n, hh, s).start()
            return 0

        lax.fori_loop(0, n_per_w, blk, 0)
        for _ in range(4):
            o_drain()

    return pl.kernel(
        body,
        out_type=jax.ShapeDtypeStruct((H, D, B), jnp.float32),
        mesh=mesh,
        compiler_params=pltpu.CompilerParams(
            needs_layout_passes=False, use_tc_tiling_on_sc=True),
        scratch_types=[
            pltpu.VMEM((2, HB, 128), jnp.int32),
            pltpu.VMEM((4, 128), jnp.int32),
            pltpu.VMEM((4, 128, 128), jnp.float32),
            pltpu.VMEM((4, D, 128), jnp.float32),
            pltpu.SemaphoreType.DMA,
            pltpu.SemaphoreType.DMA,
            pltpu.SemaphoreType.DMA,
        ],
    )


def kernel(price_hist, price_lens, tweet_hist, tweet_lens, embedding_matrix):
    tt = embedding_matrix.T
    tail = lax.slice(embedding_matrix, (NVT * 128, 0), (V, D)).reshape(32, 128)
    table2 = _detile_kernel()(tt, tail)
    idxt = tweet_hist.astype(jnp.int32).T
    out_t = _gather_kernel()(table2, idxt)
    return out_t.transpose(2, 0, 1)


# speedup vs baseline: 3.0235x; 1.5756x over previous
import jax
import jax.numpy as jnp
from jax import lax
from jax.experimental import pallas as pl
from jax.experimental.pallas import tpu as pltpu
from jax.experimental.pallas import tpu_sc as plsc

NC = 2
NS = 16
NW = NC * NS
L = 16

D = 64
V = 1000000
NVT = V // 128
VT_MAIN = NVT - (NVT % NW)
B = 4096
H = 200
HB = 8
NBT = B // 128
N_BLK = (H // HB) * NBT


def _detile_kernel():
    mesh = plsc.VectorSubcoreMesh(core_axis_name="c", subcore_axis_name="s")
    per_w = VT_MAIN // NW
    UNROLL = 4
    T = per_w // UNROLL

    def body(tt_hbm, tail_hbm, out_hbm, strip_v, rows_v, tail_v, t1_v,
             ssem, osem):
        wid = lax.axis_index("s") * NC + lax.axis_index("c")
        base = wid * per_w

        def start_strip(vt, s):
            pltpu.make_async_copy(
                tt_hbm.at[pl.ds(0, 64), pl.ds(vt * 128, 128)],
                strip_v.at[s], ssem).start()

        def wait_strip():
            pltpu.make_async_copy(
                tt_hbm.at[pl.ds(0, 64), pl.ds(0, 128)],
                strip_v.at[0], ssem).wait()

        def o_copy(vt, s):
            return pltpu.make_async_copy(
                rows_v.at[s], out_hbm.at[pl.ds(vt * 64, 64)], osem)

        def o_drain():
            pltpu.make_async_copy(
                rows_v.at[0], out_hbm.at[pl.ds(0, 64)], osem).wait()

        def transpose(s):
            iota = jnp.arange(0, L, dtype=jnp.int32)

            @plsc.parallel_loop(0, D, unroll=4)
            def _sa(d):
                for c in range(8):
                    plsc.store_scatter(
                        t1_v, [(c * L + iota) * 65 + d],
                        strip_v[s, d, pl.ds(c * L, L)])

            @plsc.parallel_loop(0, 128, unroll=4)
            def _sb(v):
                for dblk in range(D // L):
                    rows_v[s, v >> 1,
                           pl.ds((v & 1) * D + dblk * L, L)] = (
                        t1_v[pl.ds(v * 65 + dblk * L, L)])

        start_strip(base, 0)

        def step(t, _):
            for b in range(UNROLL):
                i = base + t * UNROLL + b
                sn = (b + 1) % UNROLL
                if b < UNROLL - 1:
                    @pl.when(t > 0)
                    def _w():
                        o_drain()

                    start_strip(i + 1, sn)
                else:
                    @pl.when(t + 1 < T)
                    def _wr():
                        o_drain()
                        start_strip(i + 1, sn)

                wait_strip()
                transpose(b)
                o_copy(i, b).start()
            return 0

        lax.fori_loop(0, T, step, 0)
        for _ in range(UNROLL):
            o_drain()

        @pl.when(wid < NVT - VT_MAIN)
        def _leftover():
            vt = VT_MAIN + wid
            start_strip(vt, 0)
            wait_strip()
            transpose(0)
            c = o_copy(vt, 0)
            c.start()
            c.wait()

        @pl.when(wid == NW - 1)
        def _tail():
            pltpu.sync_copy(tail_hbm, tail_v)
            pltpu.sync_copy(tail_v, out_hbm.at[pl.ds(NVT * 64, 32)])

    return pl.kernel(
        body,
        out_type=jax.ShapeDtypeStruct((V // 2, 128), jnp.float32),
        mesh=mesh,
        compiler_params=pltpu.CompilerParams(
            needs_layout_passes=False, use_tc_tiling_on_sc=True),
        scratch_types=[
            pltpu.VMEM((4, 64, 128), jnp.float32),
            pltpu.VMEM((4, 64, 128), jnp.float32),
            pltpu.VMEM((32, 128), jnp.float32),
            pltpu.VMEM((128 * 65,), jnp.float32),
            pltpu.SemaphoreType.DMA,
            pltpu.SemaphoreType.DMA,
        ],
    )


def _gather_kernel():
    mesh = plsc.VectorSubcoreMesh(core_axis_name="c", subcore_axis_name="s")
    n_per_w = N_BLK // NW

    def body(tab_hbm, idxt_hbm, out_hbm, idx_v, krows_v, g_v, o_v,
             isem, gsem, osem):
        wid = lax.axis_index("s") * NC + lax.axis_index("c")

        def item(n):
            j = wid * n_per_w + n
            hb = j // NBT
            bt = j - hb * NBT
            return hb * HB, bt * 128

        def load_idx(n, bp):
            h0, b0 = item(n)
            c = pltpu.make_async_copy(
                idxt_hbm.at[pl.ds(h0, HB), pl.ds(b0, 128)], idx_v.at[bp],
                isem)
            c.start()
            c.wait()

        def start_gather(bp, hh, s):
            for g in range(8):
                i16 = idx_v[bp, hh, pl.ds(g * L, L)]
                krows_v[s, pl.ds(g * L, L)] = jnp.right_shift(i16, 1)
            pltpu.make_async_copy(
                tab_hbm.at[krows_v.at[s]], g_v.at[s], gsem).start()

        def wait_gather():
            pltpu.make_async_copy(
                tab_hbm.at[krows_v.at[0]], g_v.at[0], gsem).wait()

        def o_copy(n, hh, s):
            h0, b0 = item(n)
            return pltpu.make_async_copy(
                o_v.at[s], out_hbm.at[h0 + hh, pl.ds(0, D), pl.ds(b0, 128)],
                osem)

        def o_drain():
            pltpu.make_async_copy(
                o_v.at[0], out_hbm.at[0, pl.ds(0, D), pl.ds(0, 128)],
                osem).wait()

        def transpose(bp, hh, s):
            for g in range(8):
                i16 = idx_v[bp, hh, pl.ds(g * L, L)]
                rows16 = jnp.arange(g * L, g * L + L, dtype=jnp.int32)
                cbase = (i16 & 1) * D
                m = (i16 != 0).astype(jnp.float32)

                @plsc.parallel_loop(0, D, unroll=8)
                def _drow(d):
                    o_v[s, d, pl.ds(g * L, L)] = plsc.load_gather(
                        g_v.at[s], [rows16, cbase + d]) * m

        load_idx(0, 0)
        start_gather(0, 0, 0)

        def blk(n, _):
            bp = n & 1
            bpn = 1 - bp
            for hh in range(HB):
                s = hh % 4
                sn = (hh + 1) % 4
                if hh < HB - 1:
                    if hh >= 3:
                        o_drain()
                    else:
                        @pl.when(n > 0)
                        def _w():
                            o_drain()
                    start_gather(bp, hh + 1, sn)
                else:
                    @pl.when(n + 1 < n_per_w)
                    def _adv():
                        o_drain()
                        load_idx(n + 1, bpn)
                        start_gather(bpn, 0, sn)

                wait_gather()
                transpose(bp, hh, s)
                o_copy(n, hh, s).start()
            return 0

        lax.fori_loop(0, n_per_w, blk, 0)
        for _ in range(4):
            o_drain()

    return pl.kernel(
        body,
        out_type=jax.ShapeDtypeStruct((H, D, B), jnp.float32),
        mesh=mesh,
        compiler_params=pltpu.CompilerParams(
            needs_layout_passes=False, use_tc_tiling_on_sc=True),
        scratch_types=[
            pltpu.VMEM((2, HB, 128), jnp.int32),
            pltpu.VMEM((4, 128), jnp.int32),
            pltpu.VMEM((4, 128, 128), jnp.float32),
            pltpu.VMEM((4, D, 128), jnp.float32),
            pltpu.SemaphoreType.DMA,
            pltpu.SemaphoreType.DMA,
            pltpu.SemaphoreType.DMA,
        ],
    )


def kernel(price_hist, price_lens, tweet_hist, tweet_lens, embedding_matrix):
    tt = embedding_matrix.T
    tail = lax.slice(embedding_matrix, (NVT * 128, 0), (V, D)).reshape(32, 128)
    table2 = _detile_kernel()(tt, tail)
    idxt = tweet_hist.astype(jnp.int32).T
    out_t = _gather_kernel()(table2, idxt)
    return out_t.transpose(2, 0, 1)


# speedup vs baseline: 4.7930x; 1.5852x over previous
import jax
import jax.numpy as jnp
from jax import lax
from jax.experimental import pallas as pl
from jax.experimental.pallas import tpu as pltpu
from jax.experimental.pallas import tpu_sc as plsc

NC = 2
NS = 16
NW = NC * NS
L = 16

D = 64
V = 1000000
NVT = V // 128
VT_MAIN = NVT - (NVT % NW)
B = 4096
H = 200
HB = 8
NBT = B // 128
N_BLK = (H // HB) * NBT


def _detile_kernel():
    mesh = plsc.VectorSubcoreMesh(core_axis_name="c", subcore_axis_name="s")
    per_w = VT_MAIN // NW
    UNROLL = 4
    T = per_w // UNROLL

    def body(tt_hbm, tail_hbm, out_hbm, strip_v, rows_v, tail_v, t1_v,
             ssem, osem):
        wid = lax.axis_index("s") * NC + lax.axis_index("c")
        base = wid * per_w

        def start_strip(vt, s):
            pltpu.make_async_copy(
                tt_hbm.at[pl.ds(0, 64), pl.ds(vt * 128, 128)],
                strip_v.at[s], ssem).start()

        def wait_strip():
            pltpu.make_async_copy(
                tt_hbm.at[pl.ds(0, 64), pl.ds(0, 128)],
                strip_v.at[0], ssem).wait()

        def o_copy(vt, s):
            return pltpu.make_async_copy(
                rows_v.at[s], out_hbm.at[pl.ds(vt * 64, 64)], osem)

        def o_drain():
            pltpu.make_async_copy(
                rows_v.at[0], out_hbm.at[pl.ds(0, 64)], osem).wait()

        def transpose(s):
            iota = jnp.arange(0, L, dtype=jnp.int32)

            @plsc.parallel_loop(0, D, unroll=4)
            def _sa(d):
                for c in range(8):
                    plsc.store_scatter(
                        t1_v, [(c * L + iota) * 65 + d],
                        strip_v[s, d, pl.ds(c * L, L)])

            @plsc.parallel_loop(0, 128, unroll=4)
            def _sb(v):
                for dblk in range(D // L):
                    rows_v[s, v >> 1,
                           pl.ds((v & 1) * D + dblk * L, L)] = (
                        t1_v[pl.ds(v * 65 + dblk * L, L)])

        start_strip(base, 0)

        def step(t, _):
            for b in range(UNROLL):
                i = base + t * UNROLL + b
                sn = (b + 1) % UNROLL
                if b < UNROLL - 1:
                    @pl.when(t > 0)
                    def _w():
                        o_drain()

                    start_strip(i + 1, sn)
                else:
                    @pl.when(t + 1 < T)
                    def _wr():
                        o_drain()
                        start_strip(i + 1, sn)

                wait_strip()
                transpose(b)
                o_copy(i, b).start()
            return 0

        lax.fori_loop(0, T, step, 0)
        for _ in range(UNROLL):
            o_drain()

        @pl.when(wid < NVT - VT_MAIN)
        def _leftover():
            vt = VT_MAIN + wid
            start_strip(vt, 0)
            wait_strip()
            transpose(0)
            c = o_copy(vt, 0)
            c.start()
            c.wait()

        @pl.when(wid == NW - 1)
        def _tail():
            pltpu.sync_copy(tail_hbm, tail_v)
            pltpu.sync_copy(tail_v, out_hbm.at[pl.ds(NVT * 64, 32)])

    return pl.kernel(
        body,
        out_type=jax.ShapeDtypeStruct((V // 2, 128), jnp.float32),
        mesh=mesh,
        compiler_params=pltpu.CompilerParams(
            needs_layout_passes=False, use_tc_tiling_on_sc=True),
        scratch_types=[
            pltpu.VMEM((4, 64, 128), jnp.float32),
            pltpu.VMEM((4, 64, 128), jnp.float32),
            pltpu.VMEM((32, 128), jnp.float32),
            pltpu.VMEM((128 * 65,), jnp.float32),
            pltpu.SemaphoreType.DMA,
            pltpu.SemaphoreType.DMA,
        ],
    )


def _gather_kernel():
    mesh = plsc.VectorSubcoreMesh(core_axis_name="c", subcore_axis_name="s")
    n_per_w = N_BLK // NW

    def body(tab_hbm, idxt_hbm, out_hbm, idx_v, krows_v, g_v, t2_v, o_v,
             isem, gsem, osem):
        wid = lax.axis_index("s") * NC + lax.axis_index("c")

        def item(n):
            j = wid * n_per_w + n
            hb = j // NBT
            bt = j - hb * NBT
            return hb * HB, bt * 128

        def load_idx(n, bp):
            h0, b0 = item(n)
            c = pltpu.make_async_copy(
                idxt_hbm.at[pl.ds(h0, HB), pl.ds(b0, 128)], idx_v.at[bp],
                isem)
            c.start()
            c.wait()

        def start_gather(bp, hh, s):
            for g in range(8):
                i16 = idx_v[bp, hh, pl.ds(g * L, L)]
                krows_v[s, pl.ds(g * L, L)] = jnp.right_shift(i16, 1)
            pltpu.make_async_copy(
                tab_hbm.at[krows_v.at[s]], g_v.at[s], gsem).start()

        def wait_gather():
            pltpu.make_async_copy(
                tab_hbm.at[krows_v.at[0]], g_v.at[0], gsem).wait()

        def o_copy(n, hh, s):
            h0, b0 = item(n)
            return pltpu.make_async_copy(
                o_v.at[s], out_hbm.at[h0 + hh, pl.ds(0, D), pl.ds(b0, 128)],
                osem)

        def o_drain():
            pltpu.make_async_copy(
                o_v.at[0], out_hbm.at[0, pl.ds(0, D), pl.ds(0, 128)],
                osem).wait()

        def transpose(bp, hh, s):
            @plsc.parallel_loop(0, 128, unroll=4)
            def _skew(j):
                for c in range(8):
                    t2_v[pl.ds(j * 129 + c * L, L)] = (
                        g_v[s, j, pl.ds(c * L, L)])

            for g in range(8):
                i16 = idx_v[bp, hh, pl.ds(g * L, L)]
                base16 = jnp.arange(g * L, g * L + L, dtype=jnp.int32) * 129
                cbase = (i16 & 1) * D
                m = (i16 != 0).astype(jnp.float32)

                @plsc.parallel_loop(0, D, unroll=8)
                def _drow(d):
                    o_v[s, d, pl.ds(g * L, L)] = plsc.load_gather(
                        t2_v, [base16 + cbase + d]) * m

        load_idx(0, 0)
        start_gather(0, 0, 0)

        def blk(n, _):
            bp = n & 1
            bpn = 1 - bp
            for hh in range(HB):
                s = hh % 4
                sn = (hh + 1) % 4
                if hh < HB - 1:
                    if hh >= 3:
                        o_drain()
                    else:
                        @pl.when(n > 0)
                        def _w():
                            o_drain()
                    start_gather(bp, hh + 1, sn)
                else:
                    @pl.when(n + 1 < n_per_w)
                    def _adv():
                        o_drain()
                        load_idx(n + 1, bpn)
                        start_gather(bpn, 0, sn)

                wait_gather()
                transpose(bp, hh, s)
                o_copy(n, hh, s).start()
            return 0

        lax.fori_loop(0, n_per_w, blk, 0)
        for _ in range(4):
            o_drain()

    return pl.kernel(
        body,
        out_type=jax.ShapeDtypeStruct((H, D, B), jnp.float32),
        mesh=mesh,
        compiler_params=pltpu.CompilerParams(
            needs_layout_passes=False, use_tc_tiling_on_sc=True),
        scratch_types=[
            pltpu.VMEM((2, HB, 128), jnp.int32),
            pltpu.VMEM((4, 128), jnp.int32),
            pltpu.VMEM((4, 128, 128), jnp.float32),
            pltpu.VMEM((128 * 129,), jnp.float32),
            pltpu.VMEM((4, D, 128), jnp.float32),
            pltpu.SemaphoreType.DMA,
            pltpu.SemaphoreType.DMA,
            pltpu.SemaphoreType.DMA,
        ],
    )


def kernel(price_hist, price_lens, tweet_hist, tweet_lens, embedding_matrix):
    tt = embedding_matrix.T
    tail = lax.slice(embedding_matrix, (NVT * 128, 0), (V, D)).reshape(32, 128)
    table2 = _detile_kernel()(tt, tail)
    idxt = tweet_hist.astype(jnp.int32).T
    out_t = _gather_kernel()(table2, idxt)
    return out_t.transpose(2, 0, 1)
